# baseline (speedup 1.0000x reference)
import jax
import jax.numpy as jnp
from jax import lax
from jax.experimental import pallas as pl
from jax.experimental.pallas import tpu as pltpu
from jax.experimental.pallas import tpu_sc as plsc

D = 64
B = 16384
NW = 32
RPW = B // NW
GRP = RPW // 16
ROWW = 24 * D
TROW = 128

CLS0 = 0
WPAY = 64
WNUM = 128
BLN = 192
ANUM = 256
C0NSPL = ANUM + 896
C1NSPL = C0NSPL + 224
C2PSPL = C1NSPL + 224
C2NSPL = C2PSPL + 16
NCONST = C2NSPL + 16
OFFC = (1, 3, 10)
PAY0 = 14


def _rsqrt16(x):
    i = plsc.bitcast(x, jnp.int32)
    i = jnp.int32(0x5F3759DF) - (i >> 1)
    y = plsc.bitcast(i, jnp.float32)
    return y * (1.5 - (x * 0.5) * y * y)


def _sc_body(tab_hbm, srep_hbm, ic_hbm, ip_hbm, c_hbm, out_hbm,
             icv, ipv, cv, ob, gbuf, sbuf, idsall, sems, sem_in):
    wid = lax.axis_index("s") * 2 + lax.axis_index("c")
    base = wid * RPW
    pltpu.sync_copy(c_hbm, cv)
    pltpu.sync_copy(ic_hbm.at[:, pl.ds(base, RPW)], icv)
    pltpu.sync_copy(ip_hbm.at[:, pl.ds(base, RPW)], ipv)

    for row in range(32):
        for k in range(4):
            ob[row, pl.ds(k * 16, 16)] = cv[pl.ds(CLS0 + 16 * k, 16)]

    def mkidx(gi, _):
        rbase = gi * 16
        for t in range(3):
            iv = icv[t, pl.ds(rbase, 16)]
            idsall[pl.ds(gi * 144 + t * 16, 16)] = iv + OFFC[t]
        for t in range(6):
            iv = ipv[t, pl.ds(rbase, 16)]
            idsall[pl.ds(gi * 144 + 48 + t * 16, 16)] = iv + (PAY0 + t * 4)
        return 0

    lax.fori_loop(0, GRP, mkidx, 0)

    def fetch(gi, p):
        pltpu.async_copy(tab_hbm.at[idsall.at[pl.ds(gi * 144, 48)]],
                         gbuf.at[pl.ds(p * 144, 48), :], sem_in.at[p])
        pltpu.async_copy(tab_hbm.at[idsall.at[pl.ds(gi * 144 + 48, 96)]],
                         gbuf.at[pl.ds(p * 144 + 48, 96), :], sem_in.at[p])
        pltpu.async_copy(srep_hbm.at[:, pl.ds((base + gi * 16) * 16, 256)],
                         sbuf.at[pl.ds(p * 24, 24), :], sem_in.at[p])

    def fwait(p):
        pltpu.make_async_copy(tab_hbm.at[pl.ds(0, 48), :],
                              gbuf.at[pl.ds(0, 48), :], sem_in.at[p]).wait()
        pltpu.make_async_copy(tab_hbm.at[pl.ds(0, 96), :],
                              gbuf.at[pl.ds(0, 96), :], sem_in.at[p]).wait()
        pltpu.make_async_copy(srep_hbm.at[:, pl.ds(0, 256)],
                              sbuf.at[pl.ds(0, 24), :], sem_in.at[p]).wait()

    c2p = cv[pl.ds(C2PSPL, 16)]
    c2n = cv[pl.ds(C2NSPL, 16)]
    wp = [cv[pl.ds(WPAY + 16 * k, 16)] for k in range(4)]
    bl = [cv[pl.ds(BLN + 16 * k, 16)] for k in range(4)]
    wn = [cv[pl.ds(WNUM + 16 * k, 16)] for k in range(4)]

    fetch(0, 0)

    def group(gi, _):
        p = gi % 2
        brow0 = p * 16

        fwait(p)

        @pl.when(gi + 1 < GRP)
        def _prefetch_next():
            fetch(gi + 1, 1 - p)

        @pl.when(gi >= 2)
        def _wait_prev():
            pltpu.make_async_copy(
                ob.at[pl.ds(brow0, 16), :],
                out_hbm.at[pl.ds(0, 16), :],
                sems.at[p]).wait()

        go = p * 144
        so = p * 24

        for t in range(3):
            def catj(j, _, t=t):
                gr = go + t * 16 + j
                brow = brow0 + j
                for k in range(4):
                    ob[brow, pl.ds((1 + t) * 64 + 16 * k, 16)] = \
                        gbuf[gr, pl.ds(16 * k, 16)]
                return 0
            lax.fori_loop(0, 16, catj, 0, unroll=4)

        for t in range(6):
            def payj(j, _, t=t):
                gr = go + 48 + t * 16 + j
                s = sbuf[so + t, pl.ds(j * 16, 16)]
                c0 = gbuf[gr, pl.ds(64, 16)]
                c1 = gbuf[gr, pl.ds(80, 16)]
                r = _rsqrt16((c2p * s + c1) * s + c0)
                brow = brow0 + j
                for k in range(4):
                    a = gbuf[gr, pl.ds(16 * k, 16)]
                    ob[brow, pl.ds((4 + t) * 64 + 16 * k, 16)] = \
                        (a + s * wp[k]) * r + bl[k]
                return 0
            lax.fori_loop(0, 16, payj, 0, unroll=4)

        for t in range(14):
            ak = [cv[pl.ds(ANUM + t * 64 + 16 * k, 16)] for k in range(4)]
            c0 = cv[pl.ds(C0NSPL + t * 16, 16)]
            c1 = cv[pl.ds(C1NSPL + t * 16, 16)]

            def numj(j, _, t=t, ak=ak, c0=c0, c1=c1):
                s = sbuf[so + 6 + t, pl.ds(j * 16, 16)]
                r = _rsqrt16((c2n * s + c1) * s + c0)
                brow = brow0 + j
                for k in range(4):
                    ob[brow, pl.ds((10 + t) * 64 + 16 * k, 16)] = \
                        (ak[k] + s * wn[k]) * r + bl[k]
                return 0
            lax.fori_loop(0, 16, numj, 0, unroll=4)

        pltpu.async_copy(
            ob.at[pl.ds(brow0, 16), :],
            out_hbm.at[pl.ds(base + gi * 16, 16), :],
            sems.at[p])
        return 0

    lax.fori_loop(0, GRP, group, 0)
    pltpu.make_async_copy(ob.at[pl.ds(0, 16), :],
                          out_hbm.at[pl.ds(0, 16), :], sems.at[0]).wait()
    pltpu.make_async_copy(ob.at[pl.ds(16, 16), :],
                          out_hbm.at[pl.ds(0, 16), :], sems.at[1]).wait()


@jax.jit
def _run_sc(tab, srep, ic, ip, consts):
    mesh = plsc.VectorSubcoreMesh(core_axis_name="c", subcore_axis_name="s",
                                  num_cores=2, num_subcores=16)
    k = pl.kernel(
        _sc_body,
        out_type=jax.ShapeDtypeStruct((B, ROWW), jnp.float32),
        mesh=mesh,
        compiler_params=pltpu.CompilerParams(needs_layout_passes=False),
        scratch_types=[
            pltpu.VMEM((3, RPW), jnp.int32),
            pltpu.VMEM((6, RPW), jnp.int32),
            pltpu.VMEM((NCONST,), jnp.float32),
            pltpu.VMEM((32, ROWW), jnp.float32),
            pltpu.VMEM((288, TROW), jnp.float32),
            pltpu.VMEM((48, 256), jnp.float32),
            pltpu.VMEM((GRP * 144,), jnp.int32),
            pltpu.SemaphoreType.DMA((2,)),
            pltpu.SemaphoreType.DMA((2,)),
        ],
    )
    return k(tab, srep, ic, ip, consts)


def kernel(cat_idx_sex, cat_idx_education, cat_idx_marriage, pay_state_ids,
           pay_severities, num_values, emb_sex, emb_education, emb_marriage,
           pay_state_table, sev_W, sev_b, num_feat_table, val_W, val_b,
           pos_table, cls_token, ln_g, ln_b):
    f32 = jnp.float32
    g = ln_g.astype(f32)
    bln = ln_b.astype(f32)
    pos = pos_table.astype(f32)
    eps = 1e-5

    rows = jnp.concatenate([
        (cls_token[0, 0] + pos[0])[None],
        emb_sex + pos[1], emb_education + pos[2], emb_marriage + pos[3],
    ], axis=0)
    mu = rows.mean(-1, keepdims=True)
    var = ((rows - mu) ** 2).mean(-1, keepdims=True)
    lncat = (rows - mu) * lax.rsqrt(var + eps) * g + bln

    w_pay = sev_W[:, 0]
    a_pay = pay_state_table[None, :, :] + sev_b + pos[4:10][:, None, :]
    ah_pay = a_pay - a_pay.mean(-1, keepdims=True)
    wh_pay = w_pay - w_pay.mean()
    c0_pay = (ah_pay ** 2).mean(-1) + eps
    c1_pay = 2.0 * (ah_pay * wh_pay).mean(-1)
    c2_pay = (wh_pay ** 2).mean()

    w_num = val_W[:, 0]
    a_num = num_feat_table + val_b + pos[10:24]
    ah_num = a_num - a_num.mean(-1, keepdims=True)
    wh_num = w_num - w_num.mean()
    c0_num = (ah_num ** 2).mean(-1) + eps
    c1_num = 2.0 * (ah_num * wh_num).mean(-1)
    c2_num = (wh_num ** 2).mean()

    cat_rows = jnp.concatenate(
        [lncat, jnp.zeros((14, 64), f32)], axis=1)
    pay_rows = jnp.concatenate([
        (ah_pay * g).reshape(24, D),
        jnp.repeat(c0_pay.reshape(24, 1), 16, axis=1),
        jnp.repeat(c1_pay.reshape(24, 1), 16, axis=1),
        jnp.zeros((24, 32), f32),
    ], axis=1)
    tab = jnp.concatenate([cat_rows, pay_rows], axis=0)

    consts = jnp.concatenate([
        lncat[0],
        wh_pay * g, wh_num * g, bln,
        (ah_num * g).reshape(-1),
        jnp.repeat(c0_num, 16), jnp.repeat(c1_num, 16),
        jnp.full((16,), c2_pay, f32), jnp.full((16,), c2_num, f32),
    ])

    s_all = jnp.concatenate([pay_severities.T, num_values.T], axis=0)
    srep = jnp.concatenate([jnp.repeat(s_all.astype(f32), 16, axis=1),
                            jnp.zeros((4, 16 * B), f32)])
    ic = jnp.stack([cat_idx_sex, cat_idx_education,
                    cat_idx_marriage]).astype(jnp.int32)
    ip = pay_state_ids.T.astype(jnp.int32)

    out = _run_sc(tab, srep, ic, ip, consts)
    return out.reshape(B, 24, D)
